# baseline (speedup 1.0000x reference)
import functools

import jax
import jax.numpy as jnp
from jax import lax
from jax.experimental import pallas as pl
from jax.experimental.pallas import tpu as pltpu
from jax.experimental.pallas import tpu_sc as plsc

N = 10000
E = 320000
D = 128
G = 64

NP_ = 10240
BM_N = 1024
BM_E = 1600

NW = 32
EPW = E // NW
NB = NP_ // NW
CHUNK = 1024
NCHUNK = EPW // CHUNK
STRIDE = 1040
ARENA = 96
GCAP = 640
BROWS = NW * NW * GCAP
SENT = NB
NEG = -1.0e30

_mesh = plsc.VectorSubcoreMesh(core_axis_name="c", subcore_axis_name="s")


def _wid():
    return lax.axis_index("s") * 2 + lax.axis_index("c")


def _vgather(v, idx):
    dn = lax.GatherDimensionNumbers(offset_dims=(), collapsed_slice_dims=(0,),
                                    start_index_map=(0,))
    return lax.gather(v, idx[:, None], dn, (1,),
                      mode=lax.GatherScatterMode.PROMISE_IN_BOUNDS)


def _mm_body(x_ref, w_ref, b_ref, o_ref, *, act_in):
    x = x_ref[...]
    if act_in:
        x = x * jax.nn.sigmoid(x)
    o_ref[...] = jnp.dot(x, w_ref[...], preferred_element_type=jnp.float32) + b_ref[...]


def _mm(x, w, b, act_in=False, bm=BM_N):
    m, k = x.shape
    c = w.shape[1]
    return pl.pallas_call(
        functools.partial(_mm_body, act_in=act_in),
        grid=(m // bm,),
        in_specs=[
            pl.BlockSpec((bm, k), lambda i: (i, 0)),
            pl.BlockSpec((k, c), lambda i: (0, 0)),
            pl.BlockSpec((1, c), lambda i: (0, 0)),
        ],
        out_specs=pl.BlockSpec((bm, c), lambda i: (i, 0)),
        out_shape=jax.ShapeDtypeStruct((m, c), jnp.float32),
    )(x, w, b)


def _tab_body(x_ref, wa_ref, wb_ref, ba_ref, oa_ref, ob_ref, *, act_in):
    x = x_ref[...]
    if act_in:
        x = x * jax.nn.sigmoid(x)
    oa_ref[...] = jnp.dot(x, wa_ref[...], preferred_element_type=jnp.float32) + ba_ref[...]
    ob_ref[...] = jnp.dot(x, wb_ref[...], preferred_element_type=jnp.float32)


def _tables(x, w0, b0, act_in=False):
    k = x.shape[1]
    wt, wb = w0[:k], w0[k:]
    wa = wt - wb
    bm = BM_N
    out_sd = jax.ShapeDtypeStruct((NP_, 32), jnp.float32)
    return pl.pallas_call(
        functools.partial(_tab_body, act_in=act_in),
        grid=(NP_ // bm,),
        in_specs=[
            pl.BlockSpec((bm, k), lambda i: (i, 0)),
            pl.BlockSpec((k, 32), lambda i: (0, 0)),
            pl.BlockSpec((k, 32), lambda i: (0, 0)),
            pl.BlockSpec((1, 32), lambda i: (0, 0)),
        ],
        out_specs=[pl.BlockSpec((bm, 32), lambda i: (i, 0)),
                   pl.BlockSpec((bm, 32), lambda i: (i, 0))],
        out_shape=[out_sd, out_sd],
    )(x, wa, wb, b0[None, :])


def _pool_body(h_ref, bt_ref, wg_ref, bg_ref, o_ref):
    h = h_ref[...]
    bt = bt_ref[...]
    gate = jnp.dot(h, wg_ref[...], preferred_element_type=jnp.float32) + bg_ref[...]
    seg = lax.broadcasted_iota(jnp.int32, (NP_, G), 1)
    onehot = (bt == seg).astype(jnp.float32)
    neg = jnp.float32(-1e30)
    gmax = jnp.max(jnp.where(onehot > 0, gate, neg), axis=0, keepdims=True)
    gmax = jnp.where(gmax <= neg, 0.0, gmax)
    mb = jnp.dot(onehot, gmax.T, preferred_element_type=jnp.float32)
    e = jnp.exp(gate - mb) * onehot.sum(axis=1, keepdims=True)
    s = lax.dot_general(onehot, e, (((0,), (0,)), ((), ())),
                        preferred_element_type=jnp.float32)
    sb = jnp.dot(onehot, s, preferred_element_type=jnp.float32)
    a = e / (sb + 1e-16)
    o_ref[...] = lax.dot_general(onehot, a * h, (((0,), (0,)), ((), ())),
                                 preferred_element_type=jnp.float32)


def _att_pool(h, batch2d, wg, bg):
    return pl.pallas_call(
        _pool_body,
        in_specs=[pl.BlockSpec((NP_, 32), lambda: (0, 0)),
                  pl.BlockSpec((NP_, 1), lambda: (0, 0)),
                  pl.BlockSpec((32, 1), lambda: (0, 0)),
                  pl.BlockSpec((1, 1), lambda: (0, 0))],
        out_specs=pl.BlockSpec((G, 32), lambda: (0, 0)),
        out_shape=jax.ShapeDtypeStruct((G, 32), jnp.float32),
    )(h, batch2d, wg, bg[None, :])


def _head_body(g_ref, w0_ref, b0_ref, w1_ref, b1_ref, w2_ref, b2_ref, o_ref):
    o = jnp.dot(g_ref[...], w0_ref[...], preferred_element_type=jnp.float32) + b0_ref[...]
    o = o * jax.nn.sigmoid(o)
    o = jnp.dot(o, w1_ref[...], preferred_element_type=jnp.float32) + b1_ref[...]
    o = o * jax.nn.sigmoid(o)
    o_ref[...] = jnp.dot(o, w2_ref[...], preferred_element_type=jnp.float32) + b2_ref[...]


def _head(g, w0, b0, w1, b1, w2, b2):
    specs = [pl.BlockSpec(a.shape, lambda nd=a.ndim: (0,) * nd)
             for a in (g, w0, b0[None, :], w1, b1[None, :], w2, b2[None, :])]
    return pl.pallas_call(
        _head_body,
        in_specs=specs,
        out_specs=pl.BlockSpec((G, 1), lambda: (0, 0)),
        out_shape=jax.ShapeDtypeStruct((G, 1), jnp.float32),
    )(g, w0, b0[None, :], w1, b1[None, :], w2, b2[None, :])


def _bucket_edges(dst):
    @functools.partial(
        pl.kernel, mesh=_mesh,
        compiler_params=pltpu.CompilerParams(needs_layout_passes=False, use_tc_tiling_on_sc=False),
        out_type=[jax.ShapeDtypeStruct((BROWS + 1, 16), jnp.int32),
                  jax.ShapeDtypeStruct((NW * NW,), jnp.int32)],
        scratch_types=[pltpu.VMEM((CHUNK,), jnp.int32),
                       pltpu.VMEM((NW * STRIDE,), jnp.int32),
                       pltpu.VMEM((ARENA, 16), jnp.int32),
                       pltpu.VMEM((ARENA,), jnp.int32),
                       pltpu.VMEM((NW,), jnp.int32),
                       pltpu.SMEM((NW,), jnp.int32),
                       pltpu.SMEM((NW,), jnp.int32),
                       pltpu.SemaphoreType.DMA],
    )
    def bucket_k(dst_hbm, blists_hbm, counts_hbm, dstc, stage, arena, idxv,
                 cntb, cnt, tot, sem):
        w = _wid()
        iota = lax.iota(jnp.int32, 16)
        lane0 = iota == 0
        sentv = jnp.full((16,), SENT, jnp.int32)

        def splat(x):
            return jnp.full((16,), x, jnp.int32)

        def init_tot(b, _):
            tot[b] = 0
            return 0
        lax.fori_loop(0, NW, init_tot, 0)

        def distribute(g, base):
            dv = dstc[pl.ds(g * 16, 16)]
            for i in range(16):
                d = dv[i]
                b = ((d >> 6) * 205) >> 10
                p = cnt[b]
                pack = (base + g * 16 + i) * 512 + (d - b * NB)
                plsc.store_scatter(stage, [splat(b * STRIDE + p)],
                                   splat(pack), mask=lane0)
                cnt[b] = p + 1

        def flush():
            aptr = 0
            for b in range(NW):
                p = cnt[b]
                r = p & 15
                plsc.store_scatter(stage, [splat(b * STRIDE + (p & -16)) + iota],
                                   sentv, mask=(iota >= r) & (r > 0))
                ngr = (p + 15) >> 4
                rowbase = (b * NW + w) * GCAP + tot[b]

                def copy_body(g2, ap):
                    row = stage[pl.ds(b * STRIDE + g2 * 16, 16)]
                    plsc.store_scatter(arena, [splat(ap), iota], row)
                    plsc.store_scatter(idxv, [splat(ap)], splat(rowbase + g2),
                                       mask=lane0)
                    return ap + 1
                aptr = lax.fori_loop(0, ngr, copy_body, aptr)
                tot[b] = tot[b] + ngr

        def do_chunk(cbase, ngroups):
            pltpu.sync_copy(dst_hbm.at[pl.ds(cbase, ngroups * 16)],
                            dstc.at[pl.ds(0, ngroups * 16)])

            def zero_cnt(b, _):
                cnt[b] = 0
                return 0
            lax.fori_loop(0, NW, zero_cnt, 0)
            for q in range(ARENA // 16):
                idxv[pl.ds(q * 16, 16)] = splat(BROWS)

            def grp_body(g, _):
                distribute(g, cbase)
                return 0
            lax.fori_loop(0, ngroups, grp_body, 0)
            flush()
            pltpu.async_copy(arena, blists_hbm.at[idxv], sem).wait()

        def chunk_body(c, _):
            do_chunk(w * EPW + c * CHUNK, CHUNK // 16)
            return 0
        lax.fori_loop(0, NCHUNK, chunk_body, 0)
        do_chunk(w * EPW + NCHUNK * CHUNK, (EPW - NCHUNK * CHUNK) // 16)

        def out_cnt(b, _):
            plsc.store_scatter(cntb, [splat(b)], splat(tot[b]), mask=lane0)
            return 0
        lax.fori_loop(0, NW, out_cnt, 0)
        pltpu.sync_copy(cntb, counts_hbm.at[pl.ds(w * NW, NW)])

    return bucket_k(dst)


def _sc_gather(ta, tb, dstv, srcv):
    nch = E // 128
    per_w = (nch + NW - 1) // NW

    @functools.partial(
        pl.kernel, mesh=_mesh,
        compiler_params=pltpu.CompilerParams(needs_layout_passes=False, use_tc_tiling_on_sc=False),
        out_type=jax.ShapeDtypeStruct((E, 32), jnp.float32),
        scratch_types=[pltpu.VMEM((128,), jnp.int32),
                       pltpu.VMEM((128,), jnp.int32),
                       pltpu.VMEM((128, 32), jnp.float32),
                       pltpu.VMEM((128, 32), jnp.float32),
                       pltpu.VMEM((128, 32), jnp.float32),
                       pltpu.SemaphoreType.DMA,
                       pltpu.SemaphoreType.DMA],
    )
    def gather_k(ta_hbm, tb_hbm, dst_hbm, src_hbm, pre_hbm,
                 dbuf, sbuf, abuf, bbuf, pbuf, sem1, sem2):
        w = _wid()

        def chunk_body(c, _):
            cid = c * NW + w

            @pl.when(cid < nch)
            def _():
                base = cid * 128
                pltpu.sync_copy(dst_hbm.at[pl.ds(base, 128)], dbuf)
                pltpu.sync_copy(src_hbm.at[pl.ds(base, 128)], sbuf)
                cp1 = pltpu.async_copy(ta_hbm.at[dbuf], abuf, sem1)
                cp2 = pltpu.async_copy(tb_hbm.at[sbuf], bbuf, sem2)
                cp1.wait()
                cp2.wait()
                for i in range(128):
                    pbuf[i, pl.ds(0, 16)] = (abuf[i, pl.ds(0, 16)]
                                             + bbuf[i, pl.ds(0, 16)])
                    pbuf[i, pl.ds(16, 16)] = (abuf[i, pl.ds(16, 16)]
                                              + bbuf[i, pl.ds(16, 16)])
                pltpu.sync_copy(pbuf, pre_hbm.at[pl.ds(base, 128)])
            return 0
        lax.fori_loop(0, per_w, chunk_body, 0)

    return gather_k(ta, tb, dstv, srcv)


def _sc_scatter_max(z, blists, counts_t):
    @functools.partial(
        pl.kernel, mesh=_mesh,
        compiler_params=pltpu.CompilerParams(needs_layout_passes=False, use_tc_tiling_on_sc=False),
        out_type=jax.ShapeDtypeStruct((NP_ * 32,), jnp.float32),
        scratch_types=[pltpu.VMEM(((NB + 1) * 32,), jnp.float32),
                       pltpu.VMEM((NW,), jnp.int32),
                       pltpu.SMEM((NW,), jnp.int32),
                       pltpu.VMEM((32, 16), jnp.int32),
                       pltpu.VMEM((512,), jnp.int32),
                       pltpu.VMEM((512,), jnp.int32),
                       pltpu.VMEM((512, 32), jnp.float32),
                       pltpu.SemaphoreType.DMA],
    )
    def scatter_k(z_hbm, bl_hbm, cnt_hbm, out_hbm, table, cnts, scnt, gbuf,
                  eidb, locb, zbuf, sem):
        w = _wid()
        negv = jnp.full((16,), NEG, jnp.float32)
        sentv = jnp.full((16,), SENT, jnp.int32)

        def init_body(r, _):
            table[pl.ds(r * 16, 16)] = negv
            return 0
        lax.fori_loop(0, (NB + 1) * 2, init_body, 0)

        pltpu.sync_copy(cnt_hbm.at[pl.ds(w * NW, NW)], cnts)
        for q in range(2):
            cv = cnts[pl.ds(q * 16, 16)]
            for i in range(16):
                scnt[q * 16 + i] = cv[i]

        def scan_body(s, _):
            ng = scnt[s]
            rowbase = (w * NW + s) * GCAP
            nch = (ng + 7) >> 3

            def chunk_body(cc, _):
                goff = rowbase + cc * 8
                grem = ng - cc * 8
                pltpu.sync_copy(bl_hbm.at[pl.ds(goff, 8)], gbuf.at[pl.ds(0, 8)])
                for g in range(8):
                    pv = gbuf[g, pl.ds(0, 16)]
                    pv = jnp.where(g < grem, pv, sentv)
                    eidb[pl.ds(g * 16, 16)] = pv >> 9
                    locb[pl.ds(g * 16, 16)] = pv & 511
                pltpu.async_copy(z_hbm.at[eidb.at[pl.ds(0, 128)]],
                                 zbuf.at[pl.ds(0, 128)], sem).wait()

                for g in range(8):
                    lv = locb[pl.ds(g * 16, 16)]
                    for i in range(16):
                        off = lv[i] * 32
                        e = g * 16 + i
                        table[pl.ds(off, 16)] = jnp.maximum(
                            table[pl.ds(off, 16)], zbuf[e, pl.ds(0, 16)])
                        table[pl.ds(off + 16, 16)] = jnp.maximum(
                            table[pl.ds(off + 16, 16)], zbuf[e, pl.ds(16, 16)])
                return 0
            lax.fori_loop(0, nch, chunk_body, 0)
            return 0
        lax.fori_loop(0, NW, scan_body, 0)

        def fin_body(r, _):
            v = table[pl.ds(r * 16, 16)]
            table[pl.ds(r * 16, 16)] = jnp.where(v == NEG, 0.0, v)
            return 0
        lax.fori_loop(0, NB * 2, fin_body, 0)
        pltpu.sync_copy(table.at[pl.ds(0, NB * 32)],
                        out_hbm.at[pl.ds(w * NB * 32, NB * 32)])

    return scatter_k(z, blists, counts_t)


def kernel(x, node_type, node_value, edge_index, batch, edge_id, c1_w0, c1_b0, c1_w1, c1_b1, c1_w2, c1_b2, c1_w3, c1_b3, g1_w, g1_b, c2_w0, c2_b0, c2_w1, c2_b1, c2_w2, c2_b2, c2_w3, c2_b3, g2_w, g2_b, h_w0, h_b0, h_w1, h_b1, h_w2, h_b2):
    src = edge_index[0]
    dst = edge_index[1]
    pad = NP_ - N
    x0 = jnp.concatenate([node_type[:, None], node_value[:, None], x], axis=1)
    x0p = jnp.pad(x0, ((0, pad), (0, 0)))
    batchp = jnp.pad(batch, (0, pad), constant_values=G)
    batch2d = batchp[:, None]

    blists, counts = _bucket_edges(dst)
    counts_t = counts.reshape(NW, NW).T.reshape(-1)

    def conv(x_in, w0, b0, w1, b1, act_in):
        ta, tb = _tables(x_in, w0, b0, act_in=act_in)
        pre = _sc_gather(ta, tb, dst, src)
        z = _mm(pre, w1, b1[None, :], act_in=True, bm=BM_E)
        return _sc_scatter_max(z, blists, counts_t).reshape(NP_, 32)

    h1 = conv(x0p, c1_w0, c1_b0, c1_w1, c1_b1, False)
    h2 = conv(h1, c1_w2, c1_b2, c1_w3, c1_b3, True)
    pooled = _att_pool(h2, batch2d, g1_w, g1_b)

    pooled_b = jnp.take(pooled, batchp, axis=0)
    x02p = jnp.concatenate([x0p, pooled_b], axis=1)
    h3 = conv(x02p, c2_w0, c2_b0, c2_w1, c2_b1, False)
    h4 = conv(h3, c2_w2, c2_b2, c2_w3, c2_b3, True)
    g = _att_pool(h4, batch2d, g2_w, g2_b)

    return _head(g, h_w0, h_b0, h_w1, h_b1, h_w2, h_b2)


# speedup vs baseline: 1.5099x; 1.5099x over previous
import functools

import jax
import jax.numpy as jnp
from jax import lax
from jax.experimental import pallas as pl
from jax.experimental.pallas import tpu as pltpu
from jax.experimental.pallas import tpu_sc as plsc

N = 10000
E = 320000
D = 128
G = 64

NP_ = 10240
BM_N = 1024
BM_E = 1600

NW = 32
EPW = E // NW
NB = NP_ // NW
CHUNK = 1024
NCHUNK = EPW // CHUNK
STRIDE = 1040
ARENA = 96
GCAP = 640
BROWS = NW * NW * GCAP
SENT = NB
NEG = -1.0e30

_mesh = plsc.VectorSubcoreMesh(core_axis_name="c", subcore_axis_name="s")


def _wid():
    return lax.axis_index("s") * 2 + lax.axis_index("c")


def _vgather(v, idx):
    dn = lax.GatherDimensionNumbers(offset_dims=(), collapsed_slice_dims=(0,),
                                    start_index_map=(0,))
    return lax.gather(v, idx[:, None], dn, (1,),
                      mode=lax.GatherScatterMode.PROMISE_IN_BOUNDS)


def _mm_body(x_ref, w_ref, b_ref, o_ref, *, act_in):
    x = x_ref[...]
    if act_in:
        x = x * jax.nn.sigmoid(x)
    o_ref[...] = jnp.dot(x, w_ref[...], preferred_element_type=jnp.float32) + b_ref[...]


def _mm(x, w, b, act_in=False, bm=BM_N):
    m, k = x.shape
    c = w.shape[1]
    return pl.pallas_call(
        functools.partial(_mm_body, act_in=act_in),
        grid=(m // bm,),
        in_specs=[
            pl.BlockSpec((bm, k), lambda i: (i, 0)),
            pl.BlockSpec((k, c), lambda i: (0, 0)),
            pl.BlockSpec((1, c), lambda i: (0, 0)),
        ],
        out_specs=pl.BlockSpec((bm, c), lambda i: (i, 0)),
        out_shape=jax.ShapeDtypeStruct((m, c), jnp.float32),
    )(x, w, b)


def _tab_body(x_ref, wa_ref, wb_ref, ba_ref, oa_ref, ob_ref, *, act_in):
    x = x_ref[...]
    if act_in:
        x = x * jax.nn.sigmoid(x)
    oa_ref[...] = jnp.dot(x, wa_ref[...], preferred_element_type=jnp.float32) + ba_ref[...]
    ob_ref[...] = jnp.dot(x, wb_ref[...], preferred_element_type=jnp.float32)


def _tables(x, w0, b0, act_in=False):
    k = x.shape[1]
    wt, wb = w0[:k], w0[k:]
    wa = wt - wb
    bm = BM_N
    out_sd = jax.ShapeDtypeStruct((NP_, 32), jnp.float32)
    return pl.pallas_call(
        functools.partial(_tab_body, act_in=act_in),
        grid=(NP_ // bm,),
        in_specs=[
            pl.BlockSpec((bm, k), lambda i: (i, 0)),
            pl.BlockSpec((k, 32), lambda i: (0, 0)),
            pl.BlockSpec((k, 32), lambda i: (0, 0)),
            pl.BlockSpec((1, 32), lambda i: (0, 0)),
        ],
        out_specs=[pl.BlockSpec((bm, 32), lambda i: (i, 0)),
                   pl.BlockSpec((bm, 32), lambda i: (i, 0))],
        out_shape=[out_sd, out_sd],
    )(x, wa, wb, b0[None, :])


def _pool_body(h_ref, bt_ref, wg_ref, bg_ref, o_ref):
    h = h_ref[...]
    bt = bt_ref[...]
    gate = jnp.dot(h, wg_ref[...], preferred_element_type=jnp.float32) + bg_ref[...]
    seg = lax.broadcasted_iota(jnp.int32, (NP_, G), 1)
    onehot = (bt == seg).astype(jnp.float32)
    neg = jnp.float32(-1e30)
    gmax = jnp.max(jnp.where(onehot > 0, gate, neg), axis=0, keepdims=True)
    gmax = jnp.where(gmax <= neg, 0.0, gmax)
    mb = jnp.dot(onehot, gmax.T, preferred_element_type=jnp.float32)
    e = jnp.exp(gate - mb) * onehot.sum(axis=1, keepdims=True)
    s = lax.dot_general(onehot, e, (((0,), (0,)), ((), ())),
                        preferred_element_type=jnp.float32)
    sb = jnp.dot(onehot, s, preferred_element_type=jnp.float32)
    a = e / (sb + 1e-16)
    o_ref[...] = lax.dot_general(onehot, a * h, (((0,), (0,)), ((), ())),
                                 preferred_element_type=jnp.float32)


def _att_pool(h, batch2d, wg, bg):
    return pl.pallas_call(
        _pool_body,
        in_specs=[pl.BlockSpec((NP_, 32), lambda: (0, 0)),
                  pl.BlockSpec((NP_, 1), lambda: (0, 0)),
                  pl.BlockSpec((32, 1), lambda: (0, 0)),
                  pl.BlockSpec((1, 1), lambda: (0, 0))],
        out_specs=pl.BlockSpec((G, 32), lambda: (0, 0)),
        out_shape=jax.ShapeDtypeStruct((G, 32), jnp.float32),
    )(h, batch2d, wg, bg[None, :])


def _head_body(g_ref, w0_ref, b0_ref, w1_ref, b1_ref, w2_ref, b2_ref, o_ref):
    o = jnp.dot(g_ref[...], w0_ref[...], preferred_element_type=jnp.float32) + b0_ref[...]
    o = o * jax.nn.sigmoid(o)
    o = jnp.dot(o, w1_ref[...], preferred_element_type=jnp.float32) + b1_ref[...]
    o = o * jax.nn.sigmoid(o)
    o_ref[...] = jnp.dot(o, w2_ref[...], preferred_element_type=jnp.float32) + b2_ref[...]


def _head(g, w0, b0, w1, b1, w2, b2):
    specs = [pl.BlockSpec(a.shape, lambda nd=a.ndim: (0,) * nd)
             for a in (g, w0, b0[None, :], w1, b1[None, :], w2, b2[None, :])]
    return pl.pallas_call(
        _head_body,
        in_specs=specs,
        out_specs=pl.BlockSpec((G, 1), lambda: (0, 0)),
        out_shape=jax.ShapeDtypeStruct((G, 1), jnp.float32),
    )(g, w0, b0[None, :], w1, b1[None, :], w2, b2[None, :])


def _bucket_edges(dst):
    @functools.partial(
        pl.kernel, mesh=_mesh,
        compiler_params=pltpu.CompilerParams(needs_layout_passes=False, use_tc_tiling_on_sc=False),
        out_type=[jax.ShapeDtypeStruct((BROWS + 1, 16), jnp.int32),
                  jax.ShapeDtypeStruct((NW * NW,), jnp.int32)],
        scratch_types=[pltpu.VMEM((CHUNK,), jnp.int32),
                       pltpu.VMEM((NW * STRIDE,), jnp.int32),
                       pltpu.VMEM((ARENA, 16), jnp.int32),
                       pltpu.VMEM((ARENA,), jnp.int32),
                       pltpu.VMEM((NW,), jnp.int32),
                       pltpu.SMEM((NW,), jnp.int32),
                       pltpu.SMEM((NW,), jnp.int32),
                       pltpu.SemaphoreType.DMA],
    )
    def bucket_k(dst_hbm, blists_hbm, counts_hbm, dstc, stage, arena, idxv,
                 cntb, cnt, tot, sem):
        w = _wid()
        iota = lax.iota(jnp.int32, 16)
        lane0 = iota == 0
        sentv = jnp.full((16,), SENT, jnp.int32)

        def splat(x):
            return jnp.full((16,), x, jnp.int32)

        def init_tot(b, _):
            tot[b] = 0
            return 0
        lax.fori_loop(0, NW, init_tot, 0)

        def distribute(g, base):
            dv = dstc[pl.ds(g * 16, 16)]
            for i in range(16):
                d = dv[i]
                b = ((d >> 6) * 205) >> 10
                p = cnt[b]
                pack = (base + g * 16 + i) * 512 + (d - b * NB)
                plsc.store_scatter(stage, [splat(b * STRIDE + p)],
                                   splat(pack), mask=lane0)
                cnt[b] = p + 1

        def flush():
            aptr = 0
            for b in range(NW):
                p = cnt[b]
                r = p & 15
                plsc.store_scatter(stage, [splat(b * STRIDE + (p & -16)) + iota],
                                   sentv, mask=(iota >= r) & (r > 0))
                ngr = (p + 15) >> 4
                rowbase = (b * NW + w) * GCAP + tot[b]

                def copy_body(g2, ap):
                    row = stage[pl.ds(b * STRIDE + g2 * 16, 16)]
                    plsc.store_scatter(arena, [splat(ap), iota], row)
                    plsc.store_scatter(idxv, [splat(ap)], splat(rowbase + g2),
                                       mask=lane0)
                    return ap + 1
                aptr = lax.fori_loop(0, ngr, copy_body, aptr)
                tot[b] = tot[b] + ngr

        def do_chunk(cbase, ngroups):
            pltpu.sync_copy(dst_hbm.at[pl.ds(cbase, ngroups * 16)],
                            dstc.at[pl.ds(0, ngroups * 16)])

            def zero_cnt(b, _):
                cnt[b] = 0
                return 0
            lax.fori_loop(0, NW, zero_cnt, 0)
            for q in range(ARENA // 16):
                idxv[pl.ds(q * 16, 16)] = splat(BROWS)

            def grp_body(g, _):
                distribute(g, cbase)
                return 0
            lax.fori_loop(0, ngroups, grp_body, 0)
            flush()
            pltpu.async_copy(arena, blists_hbm.at[idxv], sem).wait()

        def chunk_body(c, _):
            do_chunk(w * EPW + c * CHUNK, CHUNK // 16)
            return 0
        lax.fori_loop(0, NCHUNK, chunk_body, 0)
        do_chunk(w * EPW + NCHUNK * CHUNK, (EPW - NCHUNK * CHUNK) // 16)

        def out_cnt(b, _):
            plsc.store_scatter(cntb, [splat(b)], splat(tot[b]), mask=lane0)
            return 0
        lax.fori_loop(0, NW, out_cnt, 0)
        pltpu.sync_copy(cntb, counts_hbm.at[pl.ds(w * NW, NW)])

    return bucket_k(dst)


def _sc_gather(ta, tb, dstv, srcv):
    nch = E // 128
    per_w = (nch + NW - 1) // NW

    @functools.partial(
        pl.kernel, mesh=_mesh,
        compiler_params=pltpu.CompilerParams(needs_layout_passes=False, use_tc_tiling_on_sc=False),
        out_type=jax.ShapeDtypeStruct((E, 32), jnp.float32),
        scratch_types=[pltpu.VMEM((2, 128), jnp.int32),
                       pltpu.VMEM((2, 128), jnp.int32),
                       pltpu.VMEM((2, 128, 32), jnp.float32),
                       pltpu.VMEM((2, 128, 32), jnp.float32),
                       pltpu.VMEM((128, 32), jnp.float32),
                       pltpu.SemaphoreType.DMA,
                       pltpu.SemaphoreType.DMA,
                       pltpu.SemaphoreType.DMA],
    )
    def gather_k(ta_hbm, tb_hbm, dst_hbm, src_hbm, pre_hbm,
                 dbuf, sbuf, abuf, bbuf, pbuf, isem, gsem0, gsem1):
        w = _wid()
        gsems = (gsem0, gsem1)

        def valid(c):
            return c * NW + w < nch

        def fetch_idx(c, par):
            @pl.when(valid(c))
            def _():
                base = (c * NW + w) * 128
                cp1 = pltpu.async_copy(dst_hbm.at[pl.ds(base, 128)],
                                       dbuf.at[par], isem)
                cp2 = pltpu.async_copy(src_hbm.at[pl.ds(base, 128)],
                                       sbuf.at[par], isem)
                cp1.wait()
                cp2.wait()

        def fire_gathers(c, par):
            @pl.when(valid(c))
            def _():
                pltpu.async_copy(ta_hbm.at[dbuf.at[par]], abuf.at[par],
                                 gsems[par])
                pltpu.async_copy(tb_hbm.at[sbuf.at[par]], bbuf.at[par],
                                 gsems[par])

        def drain_process(c, par):
            @pl.when(valid(c))
            def _():
                base = (c * NW + w) * 128
                pltpu.make_async_copy(ta_hbm.at[dbuf.at[par]], abuf.at[par],
                                      gsems[par]).wait()
                pltpu.make_async_copy(tb_hbm.at[sbuf.at[par]], bbuf.at[par],
                                      gsems[par]).wait()
                for i in range(128):
                    pbuf[i, pl.ds(0, 16)] = (abuf[par, i, pl.ds(0, 16)]
                                             + bbuf[par, i, pl.ds(0, 16)])
                    pbuf[i, pl.ds(16, 16)] = (abuf[par, i, pl.ds(16, 16)]
                                              + bbuf[par, i, pl.ds(16, 16)])
                pltpu.sync_copy(pbuf, pre_hbm.at[pl.ds(base, 128)])

        fetch_idx(0, 0)
        fire_gathers(0, 0)

        def c2_body(c2, _):
            for p in range(2):
                c = c2 * 2 + p
                par = p
                fetch_idx(c + 1, 1 - par)
                fire_gathers(c + 1, 1 - par)
                drain_process(c, par)
            return 0
        lax.fori_loop(0, (per_w + 1) // 2, c2_body, 0)

    return gather_k(ta, tb, dstv, srcv)


def _sc_scatter_max(z, blists, counts_t):
    @functools.partial(
        pl.kernel, mesh=_mesh,
        compiler_params=pltpu.CompilerParams(needs_layout_passes=False, use_tc_tiling_on_sc=False),
        out_type=jax.ShapeDtypeStruct((NP_ * 32,), jnp.float32),
        scratch_types=[pltpu.VMEM(((NB + 1) * 32,), jnp.float32),
                       pltpu.VMEM((NW,), jnp.int32),
                       pltpu.SMEM((NW,), jnp.int32),
                       pltpu.VMEM((8, 16), jnp.int32),
                       pltpu.VMEM((128,), jnp.int32),
                       pltpu.VMEM((128,), jnp.int32),
                       pltpu.VMEM((128, 32), jnp.float32),
                       pltpu.VMEM((1, 16), jnp.int32),
                       pltpu.VMEM((16,), jnp.int32),
                       pltpu.VMEM((16, 32), jnp.float32),
                       pltpu.SemaphoreType.DMA],
    )
    def scatter_k(z_hbm, bl_hbm, cnt_hbm, out_hbm, table, cnts, scnt, gbuf,
                  eidb, locb, zbuf, gbuf1, eidb16, zbuf16, sem):
        w = _wid()
        negv = jnp.full((16,), NEG, jnp.float32)

        def init_body(r, _):
            table[pl.ds(r * 16, 16)] = negv
            return 0
        lax.fori_loop(0, (NB + 1) * 2, init_body, 0)

        pltpu.sync_copy(cnt_hbm.at[pl.ds(w * NW, NW)], cnts)
        for q in range(2):
            cv = cnts[pl.ds(q * 16, 16)]
            for i in range(16):
                scnt[q * 16 + i] = cv[i]

        def scan_body(s, _):
            ng = scnt[s]
            rowbase = (w * NW + s) * GCAP
            nfull = ng >> 3

            def chunk_body(cc, _):
                goff = rowbase + cc * 8
                pltpu.sync_copy(bl_hbm.at[pl.ds(goff, 8)], gbuf)
                for g in range(8):
                    pv = gbuf[g, pl.ds(0, 16)]
                    eidb[pl.ds(g * 16, 16)] = pv >> 9
                    locb[pl.ds(g * 16, 16)] = pv & 511
                pltpu.async_copy(z_hbm.at[eidb], zbuf, sem).wait()
                for g in range(8):
                    lv = locb[pl.ds(g * 16, 16)]
                    for i in range(16):
                        off = lv[i] * 32
                        e = g * 16 + i
                        table[pl.ds(off, 16)] = jnp.maximum(
                            table[pl.ds(off, 16)], zbuf[e, pl.ds(0, 16)])
                        table[pl.ds(off + 16, 16)] = jnp.maximum(
                            table[pl.ds(off + 16, 16)], zbuf[e, pl.ds(16, 16)])
                return 0
            lax.fori_loop(0, nfull, chunk_body, 0)

            def tail_body(t, _):
                goff = rowbase + nfull * 8 + t
                pltpu.sync_copy(bl_hbm.at[pl.ds(goff, 1)], gbuf1)
                pv = gbuf1[0, pl.ds(0, 16)]
                eidb16[pl.ds(0, 16)] = pv >> 9
                pltpu.async_copy(z_hbm.at[eidb16], zbuf16, sem).wait()
                lv = pv & 511
                for i in range(16):
                    off = lv[i] * 32
                    table[pl.ds(off, 16)] = jnp.maximum(
                        table[pl.ds(off, 16)], zbuf16[i, pl.ds(0, 16)])
                    table[pl.ds(off + 16, 16)] = jnp.maximum(
                        table[pl.ds(off + 16, 16)], zbuf16[i, pl.ds(16, 16)])
                return 0
            lax.fori_loop(0, ng - nfull * 8, tail_body, 0)
            return 0
        lax.fori_loop(0, NW, scan_body, 0)

        def fin_body(r, _):
            v = table[pl.ds(r * 16, 16)]
            table[pl.ds(r * 16, 16)] = jnp.where(v == NEG, 0.0, v)
            return 0
        lax.fori_loop(0, NB * 2, fin_body, 0)
        pltpu.sync_copy(table.at[pl.ds(0, NB * 32)],
                        out_hbm.at[pl.ds(w * NB * 32, NB * 32)])

    return scatter_k(z, blists, counts_t)


def kernel(x, node_type, node_value, edge_index, batch, edge_id, c1_w0, c1_b0, c1_w1, c1_b1, c1_w2, c1_b2, c1_w3, c1_b3, g1_w, g1_b, c2_w0, c2_b0, c2_w1, c2_b1, c2_w2, c2_b2, c2_w3, c2_b3, g2_w, g2_b, h_w0, h_b0, h_w1, h_b1, h_w2, h_b2):
    src = edge_index[0]
    dst = edge_index[1]
    pad = NP_ - N
    x0 = jnp.concatenate([node_type[:, None], node_value[:, None], x], axis=1)
    x0p = jnp.pad(x0, ((0, pad), (0, 0)))
    batchp = jnp.pad(batch, (0, pad), constant_values=G)
    batch2d = batchp[:, None]

    blists, counts = _bucket_edges(dst)
    counts_t = counts.reshape(NW, NW).T.reshape(-1)

    def conv(x_in, w0, b0, w1, b1, act_in):
        ta, tb = _tables(x_in, w0, b0, act_in=act_in)
        pre = _sc_gather(ta, tb, dst, src)
        z = _mm(pre, w1, b1[None, :], act_in=True, bm=BM_E)
        return _sc_scatter_max(z, blists, counts_t).reshape(NP_, 32)

    h1 = conv(x0p, c1_w0, c1_b0, c1_w1, c1_b1, False)
    h2 = conv(h1, c1_w2, c1_b2, c1_w3, c1_b3, True)
    pooled = _att_pool(h2, batch2d, g1_w, g1_b)

    pooled_b = jnp.take(pooled, batchp, axis=0)
    x02p = jnp.concatenate([x0p, pooled_b], axis=1)
    h3 = conv(x02p, c2_w0, c2_b0, c2_w1, c2_b1, False)
    h4 = conv(h3, c2_w2, c2_b2, c2_w3, c2_b3, True)
    g = _att_pool(h4, batch2d, g2_w, g2_b)

    return _head(g, h_w0, h_b0, h_w1, h_b1, h_w2, h_b2)
